# baseline (speedup 1.0000x reference)
import functools

import jax
import jax.numpy as jnp
from jax import lax
from jax.experimental import pallas as pl
from jax.experimental.pallas import tpu as pltpu
from jax.experimental.pallas import tpu_sc as plsc

B = 4096
N = 2048
XROW = 3 * N
L = 16
G = N // L
UNROLL = 8
RB = 8


def _sc_call(x, pe0, pe1):
    info = plsc.get_sparse_core_info()
    nc, ns = info.num_cores, info.num_subcores
    nw = nc * ns
    rows_per_w = B // nw
    blocks_per_w = rows_per_w // RB

    mesh = plsc.VectorSubcoreMesh(core_axis_name="c", subcore_axis_name="s")

    scratch = (
        [pltpu.VMEM((RB, XROW), jnp.float32) for _ in range(2)]
        + [pltpu.VMEM((5, N // 2), jnp.float32) for _ in range(2)]
        + [pltpu.VMEM((N // 2,), jnp.int32) for _ in range(2)]
        + [pltpu.VMEM((N,), jnp.float32), pltpu.VMEM((N,), jnp.float32)]
        + [pltpu.SemaphoreType.DMA for _ in range(4)]
    )

    @functools.partial(
        pl.kernel,
        mesh=mesh,
        compiler_params=pltpu.CompilerParams(needs_layout_passes=False),
        out_type=[
            jax.ShapeDtypeStruct((5, B, N), jnp.float32),
            jax.ShapeDtypeStruct((B, N), jnp.int32),
        ],
        scratch_types=scratch,
    )
    def k(x_hbm, pe0_hbm, pe1_hbm, out_hbm, mask_hbm,
          xin0, xin1, ob0, ob1, mb0, mb1, pe0v, pe1v,
          sin0, sin1, sout0, sout1):
        xin = (xin0, xin1)
        ob = (ob0, ob1)
        mb = (mb0, mb1)
        sin = (sin0, sin1)
        sout = (sout0, sout1)

        wid = lax.axis_index("s") * nc + lax.axis_index("c")
        r0 = wid * rows_per_w
        pltpu.sync_copy(pe0_hbm, pe0v)
        pltpu.sync_copy(pe1_hbm, pe1v)
        iota = lax.iota(jnp.int32, L)
        gidx = iota * 3
        zero = jnp.zeros((L,), jnp.float32)

        def issue_in(kblk, b):
            pltpu.async_copy(x_hbm.at[pl.ds(r0 + kblk * RB, RB)], xin[b], sin[b])

        def wait_in(b):
            pltpu.make_async_copy(x_hbm.at[pl.ds(0, RB)], xin[b], sin[b]).wait()

        H = N // 2

        def issue_out(r, h, b):
            pltpu.async_copy(ob[b], out_hbm.at[:, r, pl.ds(h * H, H)], sout[b])
            pltpu.async_copy(mb[b], mask_hbm.at[r, pl.ds(h * H, H)], sout[b])

        def wait_out(r, h, b):
            pltpu.make_async_copy(ob[b], out_hbm.at[:, r, pl.ds(h * H, H)],
                                  sout[b]).wait()
            pltpu.make_async_copy(mb[b], mask_hbm.at[r, pl.ds(h * H, H)],
                                  sout[b]).wait()

        def compute(bin_, r8, h, bout):
            obuf = ob[bout]
            mbuf = mb[bout]
            xbuf = xin[bin_]
            rid = jnp.full((L,), r8, jnp.int32)
            g0 = h * (G // 2)

            def grp(gi, c):
                for u in range(UNROLL):
                    gl = gi * UNROLL + u
                    g = g0 + gl
                    i0 = gidx + g * (3 * L)
                    v0 = plsc.load_gather(xbuf, [rid, i0])
                    v1 = plsc.load_gather(xbuf, [rid, i0 + 1])
                    v2 = plsc.load_gather(xbuf, [rid, i0 + 2])
                    p0 = pe0v[pl.ds(g * L, L)]
                    p1 = pe1v[pl.ds(g * L, L)]
                    m = v2 > 0.0
                    obuf[0, pl.ds(gl * L, L)] = jnp.where(m, v0, zero)
                    obuf[1, pl.ds(gl * L, L)] = jnp.where(m, v1, zero)
                    obuf[2, pl.ds(gl * L, L)] = jnp.where(m, v2, zero)
                    obuf[3, pl.ds(gl * L, L)] = jnp.where(m, p0, zero)
                    obuf[4, pl.ds(gl * L, L)] = jnp.where(m, p1, zero)
                    mbuf[pl.ds(gl * L, L)] = jnp.where(m, 1, 0)
                return c

            lax.fori_loop(0, G // 2 // UNROLL, grp, 0)

        issue_in(0, 0)
        issue_in(1, 1)

        def blk_body(jj, carry):
            for kb in range(2):
                kblk = 2 * jj + kb
                wait_in(kb)
                for r8 in range(RB):
                    i = kblk * RB + r8
                    r = r0 + i
                    for h in range(2):
                        u = 2 * i + h
                        bout = h

                        @pl.when(u >= 2)
                        def _():
                            wait_out(r - 1, h, bout)

                        compute(kb, r8, h, bout)
                        issue_out(r, h, bout)

                @pl.when(kblk + 2 < blocks_per_w)
                def _():
                    issue_in(kblk + 2, kb)

            return carry

        lax.fori_loop(0, blocks_per_w // 2, blk_body, 0)
        wait_out(r0 + rows_per_w - 1, 0, 0)
        wait_out(r0 + rows_per_w - 1, 1, 1)

    return k(x, pe0, pe1)


def kernel(x, pe):
    pe0 = jnp.asarray(pe[:, 0])
    pe1 = jnp.asarray(pe[:, 1])
    out_p, mask_i = _sc_call(x, pe0, pe1)
    return jnp.moveaxis(out_p, 0, -1), mask_i.astype(bool)


# speedup vs baseline: 1.1266x; 1.1266x over previous
import functools

import jax
import jax.numpy as jnp
from jax import lax
from jax.experimental import pallas as pl
from jax.experimental.pallas import tpu as pltpu
from jax.experimental.pallas import tpu_sc as plsc

B = 4096
N = 2048
XROW = 3 * N
L = 16
G = N // L
UNROLL = 8
NBUF = 4


def _sc_call(x, pe0, pe1):
    info = plsc.get_sparse_core_info()
    nc, ns = info.num_cores, info.num_subcores
    nw = nc * ns
    rows_per_w = B // nw

    mesh = plsc.VectorSubcoreMesh(core_axis_name="c", subcore_axis_name="s")

    scratch = (
        [pltpu.VMEM((XROW,), jnp.float32) for _ in range(NBUF)]
        + [pltpu.VMEM((5, N), jnp.float32) for _ in range(NBUF)]
        + [pltpu.VMEM((N,), jnp.int32) for _ in range(NBUF)]
        + [pltpu.VMEM((N,), jnp.float32), pltpu.VMEM((N,), jnp.float32)]
        + [pltpu.SemaphoreType.DMA for _ in range(2 * NBUF)]
    )

    @functools.partial(
        pl.kernel,
        mesh=mesh,
        compiler_params=pltpu.CompilerParams(needs_layout_passes=False),
        out_type=[
            jax.ShapeDtypeStruct((5, B, N), jnp.float32),
            jax.ShapeDtypeStruct((B, N), jnp.int32),
        ],
        scratch_types=scratch,
    )
    def k(x_hbm, pe0_hbm, pe1_hbm, out_hbm, mask_hbm, *bufs):
        xin = bufs[0:NBUF]
        ob = bufs[NBUF:2 * NBUF]
        mb = bufs[2 * NBUF:3 * NBUF]
        pe0v, pe1v = bufs[3 * NBUF], bufs[3 * NBUF + 1]
        sin = bufs[3 * NBUF + 2:3 * NBUF + 2 + NBUF]
        sout = bufs[3 * NBUF + 2 + NBUF:]

        wid = lax.axis_index("s") * nc + lax.axis_index("c")
        r0 = wid * rows_per_w
        pltpu.sync_copy(pe0_hbm, pe0v)
        pltpu.sync_copy(pe1_hbm, pe1v)
        iota = lax.iota(jnp.int32, L)
        gidx = iota * 3
        zero = jnp.zeros((L,), jnp.float32)

        def issue_in(r, b):
            pltpu.async_copy(x_hbm.at[r], xin[b], sin[b])

        def wait_in(b):
            pltpu.make_async_copy(x_hbm.at[0], xin[b], sin[b]).wait()

        def issue_out(r, b):
            pltpu.async_copy(ob[b], out_hbm.at[:, r], sout[b])
            pltpu.async_copy(mb[b], mask_hbm.at[r], sout[b])

        def wait_out(r, b):
            pltpu.make_async_copy(ob[b], out_hbm.at[:, r], sout[b]).wait()
            pltpu.make_async_copy(mb[b], mask_hbm.at[r], sout[b]).wait()

        def compute(b):
            obuf = ob[b]
            mbuf = mb[b]
            xbuf = xin[b]

            def grp(gi, c):
                for u in range(UNROLL):
                    g = gi * UNROLL + u
                    i0 = gidx + g * (3 * L)
                    v0 = plsc.load_gather(xbuf, [i0])
                    v1 = plsc.load_gather(xbuf, [i0 + 1])
                    v2 = plsc.load_gather(xbuf, [i0 + 2])
                    p0 = pe0v[pl.ds(g * L, L)]
                    p1 = pe1v[pl.ds(g * L, L)]
                    m = v2 > 0.0
                    obuf[0, pl.ds(g * L, L)] = jnp.where(m, v0, zero)
                    obuf[1, pl.ds(g * L, L)] = jnp.where(m, v1, zero)
                    obuf[2, pl.ds(g * L, L)] = jnp.where(m, v2, zero)
                    obuf[3, pl.ds(g * L, L)] = jnp.where(m, p0, zero)
                    obuf[4, pl.ds(g * L, L)] = jnp.where(m, p1, zero)
                    mbuf[pl.ds(g * L, L)] = jnp.where(m, 1, 0)
                return c

            lax.fori_loop(0, G // UNROLL, grp, 0)

        for b in range(NBUF):
            issue_in(r0 + b, b)

        def ring_body(j, carry):
            for b in range(NBUF):
                i = NBUF * j + b
                r = r0 + i
                wait_in(b)

                @pl.when(j > 0)
                def _():
                    wait_out(r - NBUF, b)

                compute(b)
                issue_out(r, b)

                @pl.when(i + NBUF < rows_per_w)
                def _():
                    issue_in(r + NBUF, b)

            return carry

        lax.fori_loop(0, rows_per_w // NBUF, ring_body, 0)
        for b in range(NBUF):
            wait_out(r0 + rows_per_w - NBUF + b, b)

    return k(x, pe0, pe1)


def kernel(x, pe):
    pe0 = jnp.asarray(pe[:, 0])
    pe1 = jnp.asarray(pe[:, 1])
    out_p, mask_i = _sc_call(x, pe0, pe1)
    return jnp.moveaxis(out_p, 0, -1), mask_i.astype(bool)
